# baseline (speedup 1.0000x reference)
import functools
from math import sqrt

import jax
import jax.numpy as jnp
from jax.experimental import pallas as pl
from jax.experimental.pallas import tpu as pltpu

LOCAL_HALF = 8
SP1 = 65


def _shifted(ref, i0, d, T, C, dtype):
    if d <= i0:
        return ref[0, i0 - d:i0 - d + T, :]
    sh = d - i0
    z = jnp.zeros((sh, C), dtype)
    return jnp.concatenate([z, ref[0, 0:T - sh, :]], axis=0)


def _body(q_ref, k_ref, v_ref, o_ref, s_ref, *, T, L, NH, D, scale):
    C = NH * D
    nblk = L // T
    f32 = jnp.float32

    ri = jax.lax.broadcasted_iota(jnp.int32, (T, T), 0)
    ci = jax.lax.broadcasted_iota(jnp.int32, (T, T), 1)
    tril = (ri >= ci).astype(f32)
    li = jax.lax.broadcasted_iota(jnp.int32, (C, NH), 0)
    hi = jax.lax.broadcasted_iota(jnp.int32, (C, NH), 1)
    sel = (li // D == hi).astype(f32)
    selt = jnp.transpose(sel)

    carry = jnp.zeros((1, C), f32)
    for blk in range(nblk):
        i0 = blk * T
        qb = q_ref[0, i0:i0 + T, :]
        vb = v_ref[0, i0:i0 + T, :]
        p = jax.lax.dot(tril, vb, preferred_element_type=f32) + carry
        carry = p[T - 1:T, :]

        diags = [d for d in range(LOCAL_HALF + 1)] + [
            SP1 * m for m in range(1, L // SP1 + 1)
            if SP1 * m <= i0 + T - 1]
        nd = len(diags)

        kb = k_ref[0, i0:i0 + T, :]
        for j, d in enumerate(diags):
            ks = kb if d == 0 else _shifted(k_ref, i0, d, T, C, f32)
            s_ref[:, NH * j:NH * (j + 1)] = jax.lax.dot(
                qb * ks, sel, preferred_element_type=f32)

        SW = s_ref.shape[1]
        s_all = s_ref[...]
        lane = jax.lax.broadcasted_iota(jnp.int32, (T, SW), 1)
        didx = lane // NH
        dval = jnp.where(didx <= LOCAL_HALF, didx,
                         SP1 * (didx - LOCAL_HALF))
        rows = jax.lax.broadcasted_iota(jnp.int32, (T, SW), 0) + i0
        w = jnp.where((rows >= dval) & (didx < nd),
                      jnp.exp(scale * s_all) - 1.0, 0.0)

        wl = jax.lax.broadcasted_iota(jnp.int32, (SW, NH), 0)
        wh = jax.lax.broadcasted_iota(jnp.int32, (SW, NH), 1)
        sumsel = (wl % NH == wh).astype(f32)
        den4 = jax.lax.dot(w, sumsel, preferred_element_type=f32)
        denb = jax.lax.dot(den4, selt, preferred_element_type=f32)
        s_ref[...] = w

        num = jnp.zeros((T, C), f32)
        for j, d in enumerate(diags):
            w4 = s_ref[:, NH * j:NH * (j + 1)]
            wb = jax.lax.dot(w4, selt, preferred_element_type=f32)
            vs = vb if d == 0 else _shifted(v_ref, i0, d, T, C, f32)
            num = num + wb * vs

        cnt = (jax.lax.broadcasted_iota(jnp.int32, (T, 1), 0)
               + (i0 + 1)).astype(f32)
        o_ref[0, i0:i0 + T, :] = (p + num) / (cnt + denb)


def kernel(queries, keys, values, attn_mask):
    B, L, H, D = queries.shape
    del attn_mask
    scale = 1.0 / sqrt(D)
    NH = 4 if H % 4 == 0 else (2 if H % 2 == 0 else 1)
    C = NH * D
    HS = H // NH
    T = 128

    qv = queries.reshape(B, L, H * D)
    kv = keys.reshape(B, L, H * D)
    vv = values.reshape(B, L, H * D)

    body = functools.partial(_body, T=T, L=L, NH=NH, D=D, scale=scale)
    out = pl.pallas_call(
        body,
        grid=(B, HS),
        in_specs=[pl.BlockSpec((1, L, C), lambda b, hs: (b, 0, hs))] * 3,
        out_specs=pl.BlockSpec((1, L, C), lambda b, hs: (b, 0, hs)),
        out_shape=jax.ShapeDtypeStruct((B, L, H * D), jnp.float32),
        scratch_shapes=[pltpu.VMEM(
            (T, NH * (LOCAL_HALF + 1 + L // SP1)), jnp.float32)],
    )(qv, kv, vv)
    return out.reshape(B, L, H, D)


# speedup vs baseline: 1.1143x; 1.1143x over previous
import functools
from math import sqrt

import jax
import jax.numpy as jnp
from jax.experimental import pallas as pl
from jax.experimental.pallas import tpu as pltpu

LOCAL_HALF = 8
SP1 = 65


def _shifted(ref, i0, d, T, C, dtype):
    if d <= i0:
        return ref[0, i0 - d:i0 - d + T, :]
    sh = d - i0
    z = jnp.zeros((sh, C), dtype)
    return jnp.concatenate([z, ref[0, 0:T - sh, :]], axis=0)


def _body(q_ref, k_ref, v_ref, o_ref, s_ref, *, T, L, NH, D, scale):
    C = NH * D
    nblk = L // T
    f32 = jnp.float32

    ri = jax.lax.broadcasted_iota(jnp.int32, (T, T), 0)
    ci = jax.lax.broadcasted_iota(jnp.int32, (T, T), 1)
    tril = (ri >= ci).astype(f32)
    li = jax.lax.broadcasted_iota(jnp.int32, (C, NH), 0)
    hi = jax.lax.broadcasted_iota(jnp.int32, (C, NH), 1)
    sel = (li // D == hi).astype(f32)
    selt = jnp.transpose(sel)

    carry = jnp.zeros((1, C), f32)
    for blk in range(nblk):
        i0 = blk * T
        qb = q_ref[0, i0:i0 + T, :]
        vb = v_ref[0, i0:i0 + T, :]
        p = jax.lax.dot(tril, vb, preferred_element_type=f32) + carry
        carry = p[T - 1:T, :]

        diags = [d for d in range(LOCAL_HALF + 1)] + [
            SP1 * m for m in range(1, L // SP1 + 1)
            if SP1 * m <= i0 + T - 1]
        nd = len(diags)

        kb = k_ref[0, i0:i0 + T, :]
        for j, d in enumerate(diags):
            ks = kb if d == 0 else _shifted(k_ref, i0, d, T, C, f32)
            s_ref[:, NH * j:NH * (j + 1)] = jax.lax.dot(
                qb * ks, sel, preferred_element_type=f32)

        SW = s_ref.shape[1]
        s_all = s_ref[...]
        lane = jax.lax.broadcasted_iota(jnp.int32, (T, SW), 1)
        didx = lane // NH
        dval = jnp.where(didx <= LOCAL_HALF, didx,
                         SP1 * (didx - LOCAL_HALF))
        rows = jax.lax.broadcasted_iota(jnp.int32, (T, SW), 0) + i0
        w = jnp.where((rows >= dval) & (didx < nd),
                      jnp.exp(scale * s_all) - 1.0, 0.0)

        wl = jax.lax.broadcasted_iota(jnp.int32, (SW, NH), 0)
        wh = jax.lax.broadcasted_iota(jnp.int32, (SW, NH), 1)
        sumsel = (wl % NH == wh).astype(f32)
        den4 = jax.lax.dot(w, sumsel, preferred_element_type=f32)
        denb = jax.lax.dot(den4, selt, preferred_element_type=f32)
        s_ref[...] = w

        num = jnp.zeros((T, C), f32)
        for j, d in enumerate(diags):
            w4 = s_ref[:, NH * j:NH * (j + 1)]
            wb = jax.lax.dot(w4, selt, preferred_element_type=f32)
            vs = vb if d == 0 else _shifted(v_ref, i0, d, T, C, f32)
            num = num + wb * vs

        cnt = (jax.lax.broadcasted_iota(jnp.int32, (T, 1), 0)
               + (i0 + 1)).astype(f32)
        o_ref[0, i0:i0 + T, :] = (p + num) / (cnt + denb)


def kernel(queries, keys, values, attn_mask):
    B, L, H, D = queries.shape
    del attn_mask
    scale = 1.0 / sqrt(D)
    NH = 4 if H % 4 == 0 else (2 if H % 2 == 0 else 1)
    C = NH * D
    HS = H // NH
    T = 512

    qv = queries.reshape(B, L, H * D)
    kv = keys.reshape(B, L, H * D)
    vv = values.reshape(B, L, H * D)

    body = functools.partial(_body, T=T, L=L, NH=NH, D=D, scale=scale)
    out = pl.pallas_call(
        body,
        grid=(B, HS),
        in_specs=[pl.BlockSpec((1, L, C), lambda b, hs: (b, 0, hs))] * 3,
        out_specs=pl.BlockSpec((1, L, C), lambda b, hs: (b, 0, hs)),
        out_shape=jax.ShapeDtypeStruct((B, L, H * D), jnp.float32),
        scratch_shapes=[pltpu.VMEM(
            (T, NH * (LOCAL_HALF + 1 + L // SP1)), jnp.float32)],
    )(qv, kv, vv)
    return out.reshape(B, L, H, D)
